# baseline (speedup 1.0000x reference)
import functools

import jax
import jax.numpy as jnp
from jax import lax
from jax.experimental import pallas as pl
from jax.experimental.pallas import tpu as pltpu
from jax.experimental.pallas import tpu_sc as plsc

_NC = 2
_NS = 16
_NW = _NC * _NS
_LANES = 16


@functools.partial(jax.jit, static_argnames=("bsz", "seq_off"))
def _sc_gather_add(idx, word_table, pos_table, bsz, seq_off=0):
    rows = idx.shape[0]
    emb = word_table.shape[1]
    rpw = rows // _NW
    ch = 32
    nch = rpw // ch
    pch = ch // bsz
    nsl = emb // _LANES

    mesh = plsc.VectorSubcoreMesh(
        core_axis_name="c", subcore_axis_name="s",
        num_cores=_NC, num_subcores=_NS)

    @functools.partial(
        pl.kernel,
        out_type=jax.ShapeDtypeStruct((rows, emb), jnp.float32),
        mesh=mesh,
        scratch_types=[
            pltpu.VMEM((rpw,), jnp.int32),
            pltpu.VMEM((ch, emb), jnp.float32),
            pltpu.VMEM((ch, emb), jnp.float32),
            pltpu.VMEM((pch, emb), jnp.float32),
            pltpu.VMEM((pch, emb), jnp.float32),
            pltpu.SemaphoreType.DMA,
            pltpu.SemaphoreType.DMA,
            pltpu.SemaphoreType.DMA,
            pltpu.SemaphoreType.DMA,
        ],
    )
    def body(idx_hbm, word_hbm, pos_hbm, out_hbm,
             idx_v, rows0, rows1, pos0, pos1, g0, g1, w0, w1):
        wid = lax.axis_index("s") * _NC + lax.axis_index("c")
        base = wid * rpw
        sbase = seq_off + wid * (rpw // bsz)
        rbuf, pbuf, gsem, wsem = (rows0, rows1), (pos0, pos1), (g0, g1), (w0, w1)

        def gather(k, buf):
            return pltpu.make_async_copy(
                word_hbm.at[idx_v.at[pl.ds(k * ch, ch)]], rbuf[buf], gsem[buf])

        def writeback(k, buf):
            return pltpu.make_async_copy(
                rbuf[buf], out_hbm.at[pl.ds(base + k * ch, ch)], wsem[buf])

        def add_pos(rows_v, pos_v):
            def fbody(p, _):
                for c in range(nsl):
                    pv = pos_v[p, pl.ds(c * _LANES, _LANES)]
                    for bb in range(bsz):
                        plsc.addupdate(
                            rows_v.at[p * bsz + bb, pl.ds(c * _LANES, _LANES)],
                            pv)
                return 0
            lax.fori_loop(0, pch, fbody, 0)

        pltpu.sync_copy(idx_hbm.at[pl.ds(base, rpw)], idx_v)
        gather(0, 0).start()
        pltpu.sync_copy(pos_hbm.at[pl.ds(sbase, pch)], pos0)
        for k in range(nch):
            cur = k % 2
            if k + 1 < nch:
                nxt = (k + 1) % 2
                if k >= 1:
                    writeback(k - 1, nxt).wait()
                gather(k + 1, nxt).start()
                pltpu.sync_copy(
                    pos_hbm.at[pl.ds(sbase + (k + 1) * pch, pch)], pbuf[nxt])
            gather(k, cur).wait()
            add_pos(rbuf[cur], pbuf[cur])
            writeback(k, cur).start()
        writeback(nch - 2, (nch - 2) % 2).wait()
        writeback(nch - 1, (nch - 1) % 2).wait()

    return body(idx, word_table, pos_table)


def _ln_matmul_body(x_ref, g_ref, bta_ref, w_ref, b_ref, o_ref):
    x = x_ref[...]
    mu = jnp.mean(x, axis=1, keepdims=True)
    xc = x - mu
    var = jnp.mean(xc * xc, axis=1, keepdims=True)
    nrm = xc * lax.rsqrt(var + 1e-5) * g_ref[...] + bta_ref[...]
    o_ref[...] = (
        jnp.dot(nrm.astype(jnp.bfloat16), w_ref[...],
                preferred_element_type=jnp.float32)
        + b_ref[...])


def _ln_matmul_chunk(prev, y, gamma, beta, w, b, part, total_rows):
    crows, emb = y.shape
    hid = w.shape[1]
    rb = 512
    nblk = crows // rb
    off = part * nblk

    def body(*refs):
        _ln_matmul_body(*refs[-6:])

    in_specs = [
        pl.BlockSpec((rb, emb), lambda i: (i, 0)),
        pl.BlockSpec((1, emb), lambda i: (0, 0)),
        pl.BlockSpec((1, emb), lambda i: (0, 0)),
        pl.BlockSpec((emb, hid), lambda i: (0, 0)),
        pl.BlockSpec((1, hid), lambda i: (0, 0)),
    ]
    args = (y, gamma, beta, w, b)
    kwargs = {}
    if prev is not None:
        in_specs = [pl.BlockSpec(memory_space=pl.ANY)] + in_specs
        args = (prev,) + args
        kwargs["input_output_aliases"] = {0: 0}
    return pl.pallas_call(
        body,
        grid=(nblk,),
        in_specs=in_specs,
        out_specs=pl.BlockSpec((rb, hid), lambda i: (i + off, 0)),
        out_shape=jax.ShapeDtypeStruct((total_rows, hid), jnp.float32),
        **kwargs,
    )(*args)


def kernel(input_ids, word_table, pos_table, gamma, beta, W, b):
    seq, bsz = input_ids.shape
    vocab, emb = word_table.shape
    hid = W.shape[1]
    idx = input_ids.reshape(-1).astype(jnp.int32)
    rows = seq * bsz
    nchunks = 4
    crows = rows // nchunks
    cseq = seq // nchunks
    wb = W.astype(jnp.bfloat16)
    g2, bta2, b2 = gamma.reshape(1, emb), beta.reshape(1, emb), b.reshape(1, hid)

    ys = [_sc_gather_add(idx[p * crows:(p + 1) * crows], word_table,
                         pos_table, bsz, seq_off=p * cseq)
          for p in range(nchunks)]
    out = None
    for p in range(nchunks):
        out = _ln_matmul_chunk(out, ys[p], g2, bta2, wb, b2, p, rows)
    return out.reshape(seq, bsz, hid)


# speedup vs baseline: 1.0084x; 1.0084x over previous
import functools

import jax
import jax.numpy as jnp
from jax import lax
from jax.experimental import pallas as pl
from jax.experimental.pallas import tpu as pltpu
from jax.experimental.pallas import tpu_sc as plsc

_NC = 2
_NS = 16
_NW = _NC * _NS
_LANES = 16


@functools.partial(jax.jit, static_argnames=("bsz", "seq_off"))
def _sc_gather_add(idx, word_table, pos_table, bsz, seq_off=0):
    rows = idx.shape[0]
    emb = word_table.shape[1]
    rpw = rows // _NW
    ch = 32
    nch = rpw // ch
    pch = ch // bsz
    nsl = emb // _LANES

    mesh = plsc.VectorSubcoreMesh(
        core_axis_name="c", subcore_axis_name="s",
        num_cores=_NC, num_subcores=_NS)

    @functools.partial(
        pl.kernel,
        out_type=jax.ShapeDtypeStruct((rows, emb), jnp.float32),
        mesh=mesh,
        scratch_types=[
            pltpu.VMEM((rpw,), jnp.int32),
            pltpu.VMEM((ch, emb), jnp.float32),
            pltpu.VMEM((ch, emb), jnp.float32),
            pltpu.VMEM((pch, emb), jnp.float32),
            pltpu.VMEM((pch, emb), jnp.float32),
            pltpu.SemaphoreType.DMA,
            pltpu.SemaphoreType.DMA,
            pltpu.SemaphoreType.DMA,
            pltpu.SemaphoreType.DMA,
        ],
    )
    def body(idx_hbm, word_hbm, pos_hbm, out_hbm,
             idx_v, rows0, rows1, pos0, pos1, g0, g1, w0, w1):
        wid = lax.axis_index("s") * _NC + lax.axis_index("c")
        base = wid * rpw
        sbase = seq_off + wid * (rpw // bsz)
        rbuf, pbuf, gsem, wsem = (rows0, rows1), (pos0, pos1), (g0, g1), (w0, w1)

        def gather(k, buf):
            return pltpu.make_async_copy(
                word_hbm.at[idx_v.at[pl.ds(k * ch, ch)]], rbuf[buf], gsem[buf])

        def writeback(k, buf):
            return pltpu.make_async_copy(
                rbuf[buf], out_hbm.at[pl.ds(base + k * ch, ch)], wsem[buf])

        def add_pos(rows_v, pos_v):
            def fbody(p, _):
                for c in range(nsl):
                    pv = pos_v[p, pl.ds(c * _LANES, _LANES)]
                    for bb in range(bsz):
                        plsc.addupdate(
                            rows_v.at[p * bsz + bb, pl.ds(c * _LANES, _LANES)],
                            pv)
                return 0
            lax.fori_loop(0, pch, fbody, 0)

        pltpu.sync_copy(idx_hbm.at[pl.ds(base, rpw)], idx_v)
        gather(0, 0).start()
        pltpu.sync_copy(pos_hbm.at[pl.ds(sbase, pch)], pos0)
        for k in range(nch):
            cur = k % 2
            if k + 1 < nch:
                nxt = (k + 1) % 2
                if k >= 1:
                    writeback(k - 1, nxt).wait()
                gather(k + 1, nxt).start()
                pltpu.sync_copy(
                    pos_hbm.at[pl.ds(sbase + (k + 1) * pch, pch)], pbuf[nxt])
            gather(k, cur).wait()
            add_pos(rbuf[cur], pbuf[cur])
            writeback(k, cur).start()
        writeback(nch - 2, (nch - 2) % 2).wait()
        writeback(nch - 1, (nch - 1) % 2).wait()

    return body(idx, word_table, pos_table)


def _ln_matmul_body(x_ref, g_ref, bta_ref, w_ref, b_ref, o_ref):
    x = x_ref[...]
    mu = jnp.mean(x, axis=1, keepdims=True)
    xc = x - mu
    var = jnp.mean(xc * xc, axis=1, keepdims=True)
    nrm = xc * lax.rsqrt(var + 1e-5) * g_ref[...] + bta_ref[...]
    o_ref[...] = (
        jnp.dot(nrm.astype(jnp.bfloat16), w_ref[...],
                preferred_element_type=jnp.float32)
        + b_ref[...])


def _ln_matmul_chunk(prev, y, gamma, beta, w, b, part, total_rows):
    crows, emb = y.shape
    hid = w.shape[1]
    rb = 1024
    nblk = crows // rb
    off = part * nblk

    def body(*refs):
        _ln_matmul_body(*refs[-6:])

    in_specs = [
        pl.BlockSpec((rb, emb), lambda i: (i, 0)),
        pl.BlockSpec((1, emb), lambda i: (0, 0)),
        pl.BlockSpec((1, emb), lambda i: (0, 0)),
        pl.BlockSpec((emb, hid), lambda i: (0, 0)),
        pl.BlockSpec((1, hid), lambda i: (0, 0)),
    ]
    args = (y, gamma, beta, w, b)
    kwargs = {}
    if prev is not None:
        in_specs = [pl.BlockSpec(memory_space=pl.ANY)] + in_specs
        args = (prev,) + args
        kwargs["input_output_aliases"] = {0: 0}
    return pl.pallas_call(
        body,
        grid=(nblk,),
        in_specs=in_specs,
        out_specs=pl.BlockSpec((rb, hid), lambda i: (i + off, 0)),
        out_shape=jax.ShapeDtypeStruct((total_rows, hid), jnp.float32),
        **kwargs,
    )(*args)


def kernel(input_ids, word_table, pos_table, gamma, beta, W, b):
    seq, bsz = input_ids.shape
    vocab, emb = word_table.shape
    hid = W.shape[1]
    idx = input_ids.reshape(-1).astype(jnp.int32)
    rows = seq * bsz
    nchunks = 2
    crows = rows // nchunks
    cseq = seq // nchunks
    wb = W.astype(jnp.bfloat16)
    g2, bta2, b2 = gamma.reshape(1, emb), beta.reshape(1, emb), b.reshape(1, hid)

    ys = [_sc_gather_add(idx[p * crows:(p + 1) * crows], word_table,
                         pos_table, bsz, seq_off=p * cseq)
          for p in range(nchunks)]
    out = None
    for p in range(nchunks):
        out = _ln_matmul_chunk(out, ys[p], g2, bta2, wb, b2, p, rows)
    return out.reshape(seq, bsz, hid)


# speedup vs baseline: 1.0349x; 1.0262x over previous
import functools

import jax
import jax.numpy as jnp
from jax import lax
from jax.experimental import pallas as pl
from jax.experimental.pallas import tpu as pltpu
from jax.experimental.pallas import tpu_sc as plsc

_NC = 2
_NS = 16
_NW = _NC * _NS
_LANES = 16


@functools.partial(jax.jit, static_argnames=("bsz", "seq_off"))
def _sc_gather_add(idx, word_table, pos_table, bsz, seq_off=0):
    rows = idx.shape[0]
    emb = word_table.shape[1]
    rpw = rows // _NW
    ch = 32
    nch = rpw // ch
    pch = ch // bsz
    nsl = emb // _LANES

    mesh = plsc.VectorSubcoreMesh(
        core_axis_name="c", subcore_axis_name="s",
        num_cores=_NC, num_subcores=_NS)

    @functools.partial(
        pl.kernel,
        out_type=jax.ShapeDtypeStruct((rows, emb), jnp.float32),
        mesh=mesh,
        scratch_types=[
            pltpu.VMEM((rpw,), jnp.int32),
            pltpu.VMEM((ch, emb), jnp.float32),
            pltpu.VMEM((ch, emb), jnp.float32),
            pltpu.VMEM((pch, emb), jnp.float32),
            pltpu.VMEM((pch, emb), jnp.float32),
            pltpu.SemaphoreType.DMA,
            pltpu.SemaphoreType.DMA,
            pltpu.SemaphoreType.DMA,
            pltpu.SemaphoreType.DMA,
        ],
    )
    def body(idx_hbm, word_hbm, pos_hbm, out_hbm,
             idx_v, rows0, rows1, pos0, pos1, g0, g1, w0, w1):
        wid = lax.axis_index("s") * _NC + lax.axis_index("c")
        base = wid * rpw
        sbase = seq_off + wid * (rpw // bsz)
        rbuf, pbuf, gsem, wsem = (rows0, rows1), (pos0, pos1), (g0, g1), (w0, w1)

        def gather(k, buf):
            return pltpu.make_async_copy(
                word_hbm.at[idx_v.at[pl.ds(k * ch, ch)]], rbuf[buf], gsem[buf])

        def writeback(k, buf):
            return pltpu.make_async_copy(
                rbuf[buf], out_hbm.at[pl.ds(base + k * ch, ch)], wsem[buf])

        def add_pos(rows_v, pos_v):
            def fbody(p, _):
                for c in range(nsl):
                    pv = pos_v[p, pl.ds(c * _LANES, _LANES)]
                    for bb in range(bsz):
                        plsc.addupdate(
                            rows_v.at[p * bsz + bb, pl.ds(c * _LANES, _LANES)],
                            pv)
                return 0
            lax.fori_loop(0, pch, fbody, 0)

        pltpu.sync_copy(idx_hbm.at[pl.ds(base, rpw)], idx_v)
        gather(0, 0).start()
        pltpu.sync_copy(pos_hbm.at[pl.ds(sbase, pch)], pos0)
        for k in range(nch):
            cur = k % 2
            if k + 1 < nch:
                nxt = (k + 1) % 2
                if k >= 1:
                    writeback(k - 1, nxt).wait()
                gather(k + 1, nxt).start()
                pltpu.sync_copy(
                    pos_hbm.at[pl.ds(sbase + (k + 1) * pch, pch)], pbuf[nxt])
            gather(k, cur).wait()
            add_pos(rbuf[cur], pbuf[cur])
            writeback(k, cur).start()
        writeback(nch - 2, (nch - 2) % 2).wait()
        writeback(nch - 1, (nch - 1) % 2).wait()

    return body(idx, word_table, pos_table)


def _ln_matmul_body(x_ref, g_ref, bta_ref, w_ref, b_ref, o_ref):
    x = x_ref[...]
    mu = jnp.mean(x, axis=1, keepdims=True)
    xc = x - mu
    var = jnp.mean(xc * xc, axis=1, keepdims=True)
    nrm = xc * lax.rsqrt(var + 1e-5) * g_ref[...] + bta_ref[...]
    o_ref[...] = (
        jnp.dot(nrm.astype(jnp.bfloat16), w_ref[...],
                preferred_element_type=jnp.float32)
        + b_ref[...])


def _ln_matmul_chunk(prev, y, gamma, beta, w, b, part, total_rows):
    crows, emb = y.shape
    hid = w.shape[1]
    rb = 1024
    nblk = crows // rb
    off = part * nblk

    def body(*refs):
        _ln_matmul_body(*refs[-6:])

    in_specs = [
        pl.BlockSpec((rb, emb), lambda i: (i, 0)),
        pl.BlockSpec((1, emb), lambda i: (0, 0)),
        pl.BlockSpec((1, emb), lambda i: (0, 0)),
        pl.BlockSpec((emb, hid), lambda i: (0, 0)),
        pl.BlockSpec((1, hid), lambda i: (0, 0)),
    ]
    args = (y, gamma, beta, w, b)
    kwargs = {}
    if prev is not None:
        in_specs = [pl.BlockSpec(memory_space=pl.ANY)] + in_specs
        args = (prev,) + args
        kwargs["input_output_aliases"] = {0: 0}
    return pl.pallas_call(
        body,
        grid=(nblk,),
        in_specs=in_specs,
        out_specs=pl.BlockSpec((rb, hid), lambda i: (i + off, 0)),
        out_shape=jax.ShapeDtypeStruct((total_rows, hid), jnp.float32),
        **kwargs,
    )(*args)


def kernel(input_ids, word_table, pos_table, gamma, beta, W, b):
    seq, bsz = input_ids.shape
    vocab, emb = word_table.shape
    hid = W.shape[1]
    idx = input_ids.reshape(-1).astype(jnp.int32)
    rows = seq * bsz
    nchunks = 4
    crows = rows // nchunks
    cseq = seq // nchunks
    wb = W.astype(jnp.bfloat16)
    g2, bta2, b2 = gamma.reshape(1, emb), beta.reshape(1, emb), b.reshape(1, hid)

    ys = [_sc_gather_add(idx[p * crows:(p + 1) * crows], word_table,
                         pos_table, bsz, seq_off=p * cseq)
          for p in range(nchunks)]
    out = None
    for p in range(nchunks):
        out = _ln_matmul_chunk(out, ys[p], g2, bta2, wb, b2, p, rows)
    return out.reshape(seq, bsz, hid)
